# baseline (speedup 1.0000x reference)
import functools

import jax
import jax.numpy as jnp
from jax import lax
from jax.experimental import pallas as pl
from jax.experimental.pallas import tpu as pltpu
from jax.experimental.pallas import tpu_sc as plsc

_BM = 128


def _route_kern(opt_ref, invp_ref, bounds_ref):
    opt = opt_ref[...]
    n_b, n_o = opt.shape
    e_iota = lax.broadcasted_iota(jnp.int32, (n_b, n_o), 1)
    oh = (opt == e_iota).astype(jnp.float32)
    gsz = _BM
    ng = n_b // gsz
    r = lax.broadcasted_iota(jnp.int32, (gsz, gsz), 0)
    c = lax.broadcasted_iota(jnp.int32, (gsz, gsz), 1)
    tri_incl = (r >= c).astype(jnp.float32)
    offs = jnp.zeros((1, n_o), jnp.float32)
    rank_excl = []
    for g in range(ng):
        oh_g = oh[g * gsz:(g + 1) * gsz, :]
        incl_g = jnp.dot(tri_incl, oh_g, preferred_element_type=jnp.float32)
        rank_excl.append(incl_g + offs - oh_g)
        offs = offs + incl_g[gsz - 1:, :]
    counts = offs
    r2 = lax.broadcasted_iota(jnp.int32, (n_o, n_o), 0)
    c2 = lax.broadcasted_iota(jnp.int32, (n_o, n_o), 1)
    strict_lt = (r2 < c2).astype(jnp.float32)
    starts = jnp.dot(counts, strict_lt,
                     preferred_element_type=jnp.float32)
    for g in range(ng):
        oh_g = oh[g * gsz:(g + 1) * gsz, :]
        pos_g = jnp.sum(oh_g * (starts + rank_excl[g]), axis=1, keepdims=True)
        invp_ref[g * gsz:(g + 1) * gsz, :] = pos_g.astype(jnp.int32)
    bounds_ref[...] = jnp.concatenate(
        [starts, starts + counts], axis=0).astype(jnp.int32)


def _tc_route(opt_bcast):
    n_b, n_o = opt_bcast.shape
    return pl.pallas_call(
        _route_kern,
        out_shape=(jax.ShapeDtypeStruct((n_b, 1), jnp.int32),
                   jax.ShapeDtypeStruct((2, n_o), jnp.int32)),
    )(opt_bcast)


def _mlp_kern(bnd_ref, xs_ref, invp_ref, l1_ref, l2_ref, ml_ref, lsl_ref,
              mb_ref, lsb_ref, out_ref, acc_ref):
    i = pl.program_id(0)
    e = pl.program_id(1)
    nb = pl.num_programs(0) - 1
    n_o = l1_ref.shape[0]
    n_b = acc_ref.shape[0]
    a = mb_ref.shape[2]
    s = bnd_ref[e]
    t = bnd_ref[n_o + e]
    lo = i * _BM

    @pl.when((i < nb) & (s < lo + _BM) & (t > lo))
    def _():
        x = xs_ref[...]
        h1 = jnp.maximum(
            jnp.dot(x, l1_ref[e], preferred_element_type=jnp.float32), 0.0)
        h2 = jnp.maximum(
            jnp.dot(h1, l2_ref[e], preferred_element_type=jnp.float32), 0.0)
        m = jnp.dot(h2, ml_ref[e], preferred_element_type=jnp.float32)
        m = m + mb_ref[e]
        l = jnp.dot(h2, lsl_ref[e], preferred_element_type=jnp.float32)
        l = l + lsb_ref[e]
        l = jnp.clip(l, -20.0, 2.0)
        rows = lo + lax.broadcasted_iota(jnp.int32, (_BM, 2 * a), 0)
        mask = (rows >= s) & (rows < t)
        val = jnp.concatenate([m, l], axis=1)
        win = acc_ref[pl.ds(lo, _BM), :]
        acc_ref[pl.ds(lo, _BM), :] = jnp.where(mask, val, win)

    @pl.when(i == nb)
    def _():
        ob = out_ref.shape[0]
        invp_blk = invp_ref[...]
        cols = lax.broadcasted_iota(jnp.int32, (ob, n_b), 1)
        oh = (invp_blk == cols).astype(jnp.float32)
        out_ref[...] = jnp.dot(oh, acc_ref[...],
                               preferred_element_type=jnp.float32)


def _tc_mlp(bounds_flat, xs, invp, linear1, linear2, ml, lsl, mb3, lsb3):
    n_b, n_i = xs.shape
    n_o, _, n_h = linear1.shape
    hc = linear2.shape[2]
    a = mb3.shape[2]
    nb = n_b // _BM
    ob = n_b // n_o
    grid_spec = pltpu.PrefetchScalarGridSpec(
        num_scalar_prefetch=1,
        grid=(nb + 1, n_o),
        in_specs=[
            pl.BlockSpec((_BM, n_i), lambda i, e, b: (jnp.minimum(i, nb - 1), 0)),
            pl.BlockSpec((ob, 1), lambda i, e, b: (e, 0)),
            pl.BlockSpec((n_o, n_i, n_h), lambda i, e, b: (0, 0, 0)),
            pl.BlockSpec((n_o, n_h, hc), lambda i, e, b: (0, 0, 0)),
            pl.BlockSpec((n_o, hc, a), lambda i, e, b: (0, 0, 0)),
            pl.BlockSpec((n_o, hc, a), lambda i, e, b: (0, 0, 0)),
            pl.BlockSpec((n_o, 1, a), lambda i, e, b: (0, 0, 0)),
            pl.BlockSpec((n_o, 1, a), lambda i, e, b: (0, 0, 0)),
        ],
        out_specs=pl.BlockSpec((ob, 2 * a), lambda i, e, b: ((i // nb) * e, 0)),
        scratch_shapes=[pltpu.VMEM((n_b, 2 * a), jnp.float32)],
    )
    return pl.pallas_call(
        _mlp_kern,
        grid_spec=grid_spec,
        out_shape=jax.ShapeDtypeStruct((n_b, 2 * a), jnp.float32),
        compiler_params=pltpu.CompilerParams(
            dimension_semantics=("arbitrary", "arbitrary")),
    )(bounds_flat, xs, invp, linear1, linear2, ml, lsl, mb3, lsb3)


def _sc_scatter_rows(data, invp_row):
    n_b, d = data.shape
    info = plsc.get_sparse_core_info()
    nw = info.num_cores * info.num_subcores
    bw = n_b // nw
    mesh = plsc.VectorSubcoreMesh(core_axis_name="c", subcore_axis_name="s")

    @functools.partial(
        pl.kernel, mesh=mesh,
        out_type=jax.ShapeDtypeStruct((n_b, d), jnp.float32),
        scratch_types=[pltpu.VMEM((bw,), jnp.int32),
                       pltpu.VMEM((bw, d), jnp.float32),
                       pltpu.SemaphoreType.DMA])
    def scatter_k(data_hbm, invp_hbm, out_hbm, idx_v, rows_v, sem):
        wid = lax.axis_index("s") * info.num_cores + lax.axis_index("c")
        base = wid * bw
        pltpu.sync_copy(invp_hbm.at[pl.ds(base, bw)], idx_v)
        pltpu.sync_copy(data_hbm.at[pl.ds(base, bw)], rows_v)
        pltpu.async_copy(rows_v, out_hbm.at[idx_v], sem).wait()

    return scatter_k(data, invp_row)


def kernel(state, option, linear1, linear2, mean_linear, log_std_linear,
           mean_bias, log_std_bias):
    n_b, _ = state.shape
    n_o = linear1.shape[0]
    a = mean_bias.shape[1]
    opt_bcast = jnp.broadcast_to(
        option.astype(jnp.int32).reshape(n_b, 1), (n_b, n_o))
    mb3 = mean_bias.reshape(n_o, 1, a)
    lsb3 = log_std_bias.reshape(n_o, 1, a)

    invp, bounds = _tc_route(opt_bcast)
    invp_row = invp.reshape(n_b)
    bounds_flat = bounds.reshape(2 * n_o)

    sorted_state = _sc_scatter_rows(state, invp_row)
    final = _tc_mlp(bounds_flat, sorted_state, invp, linear1, linear2,
                    mean_linear, log_std_linear, mb3, lsb3)
    return final[:, :a], final[:, a:2 * a]


# speedup vs baseline: 2.1709x; 2.1709x over previous
import functools

import jax
import jax.numpy as jnp
from jax import lax
from jax.experimental import pallas as pl
from jax.experimental.pallas import tpu as pltpu
from jax.experimental.pallas import tpu_sc as plsc

_BM = 128


def _route_kern(opt_ref, invp_ref, bounds_ref):
    opt = opt_ref[...]
    n_b, n_o = opt.shape
    e_iota = lax.broadcasted_iota(jnp.int32, (n_b, n_o), 1)
    oh = (opt == e_iota).astype(jnp.float32)
    gsz = _BM
    ng = n_b // gsz
    r = lax.broadcasted_iota(jnp.int32, (gsz, gsz), 0)
    c = lax.broadcasted_iota(jnp.int32, (gsz, gsz), 1)
    tri_incl = (r >= c).astype(jnp.float32)
    offs = jnp.zeros((1, n_o), jnp.float32)
    rank_excl = []
    for g in range(ng):
        oh_g = oh[g * gsz:(g + 1) * gsz, :]
        incl_g = jnp.dot(tri_incl, oh_g, preferred_element_type=jnp.float32)
        rank_excl.append(incl_g + offs - oh_g)
        offs = offs + incl_g[gsz - 1:, :]
    counts = offs
    r2 = lax.broadcasted_iota(jnp.int32, (n_o, n_o), 0)
    c2 = lax.broadcasted_iota(jnp.int32, (n_o, n_o), 1)
    strict_lt = (r2 < c2).astype(jnp.float32)
    starts = jnp.dot(counts, strict_lt,
                     preferred_element_type=jnp.float32)
    for g in range(ng):
        oh_g = oh[g * gsz:(g + 1) * gsz, :]
        pos_g = jnp.sum(oh_g * (starts + rank_excl[g]), axis=1, keepdims=True)
        invp_ref[g * gsz:(g + 1) * gsz, :] = pos_g.astype(jnp.int32)
    bounds_ref[...] = jnp.concatenate(
        [starts, starts + counts], axis=0).astype(jnp.int32)


def _tc_route(opt_bcast):
    n_b, n_o = opt_bcast.shape
    return pl.pallas_call(
        _route_kern,
        out_shape=(jax.ShapeDtypeStruct((n_b, 1), jnp.int32),
                   jax.ShapeDtypeStruct((2, n_o), jnp.int32)),
    )(opt_bcast)


def _mlp_kern(bnd_ref, xs_ref, invp_ref, l1_ref, l2_ref, ml_ref, lsl_ref,
              mb_ref, lsb_ref, out_ref, acc_ref):
    i = pl.program_id(0)
    nb = pl.num_programs(0) - 1
    n_o = l1_ref.shape[0]
    n_b = acc_ref.shape[0]
    a = mb_ref.shape[2]
    lo = i * _BM

    @pl.when(i < nb)
    def _():
        def count_lo(e, acc):
            return acc + jnp.where(bnd_ref[n_o + e] <= lo, 1, 0)

        def count_hi(e, acc):
            return acc + jnp.where(bnd_ref[e] < lo + _BM, 1, 0)

        e_lo = lax.fori_loop(0, n_o, count_lo, 0)
        e_hi = lax.fori_loop(0, n_o, count_hi, 0)
        x = xs_ref[...]
        rows = lo + lax.broadcasted_iota(jnp.int32, (_BM, 2 * a), 0)

        def body(e, _):
            s = bnd_ref[e]
            t = bnd_ref[n_o + e]
            h1 = jnp.maximum(
                jnp.dot(x, l1_ref[e], preferred_element_type=jnp.float32),
                0.0)
            h2 = jnp.maximum(
                jnp.dot(h1, l2_ref[e], preferred_element_type=jnp.float32),
                0.0)
            m = jnp.dot(h2, ml_ref[e], preferred_element_type=jnp.float32)
            m = m + mb_ref[e]
            l = jnp.dot(h2, lsl_ref[e], preferred_element_type=jnp.float32)
            l = l + lsb_ref[e]
            l = jnp.clip(l, -20.0, 2.0)
            mask = (rows >= s) & (rows < t)
            val = jnp.concatenate([m, l], axis=1)
            win = acc_ref[pl.ds(lo, _BM), :]
            acc_ref[pl.ds(lo, _BM), :] = jnp.where(mask, val, win)
            return 0

        lax.fori_loop(e_lo, e_hi, body, 0)

    @pl.when(i == nb)
    def _():
        invp_col = invp_ref[...]
        cols = lax.broadcasted_iota(jnp.int32, (n_b, n_b), 1)
        oh = (invp_col == cols).astype(jnp.float32)
        out_ref[...] = jnp.dot(oh, acc_ref[...],
                               preferred_element_type=jnp.float32)


def _tc_mlp(bounds_flat, xs, invp, linear1, linear2, ml, lsl, mb3, lsb3):
    n_b, n_i = xs.shape
    n_o, _, n_h = linear1.shape
    hc = linear2.shape[2]
    a = mb3.shape[2]
    nb = n_b // _BM
    grid_spec = pltpu.PrefetchScalarGridSpec(
        num_scalar_prefetch=1,
        grid=(nb + 1,),
        in_specs=[
            pl.BlockSpec((_BM, n_i), lambda i, b: (jnp.minimum(i, nb - 1), 0)),
            pl.BlockSpec((n_b, 1), lambda i, b: (0, 0)),
            pl.BlockSpec((n_o, n_i, n_h), lambda i, b: (0, 0, 0)),
            pl.BlockSpec((n_o, n_h, hc), lambda i, b: (0, 0, 0)),
            pl.BlockSpec((n_o, hc, a), lambda i, b: (0, 0, 0)),
            pl.BlockSpec((n_o, hc, a), lambda i, b: (0, 0, 0)),
            pl.BlockSpec((n_o, 1, a), lambda i, b: (0, 0, 0)),
            pl.BlockSpec((n_o, 1, a), lambda i, b: (0, 0, 0)),
        ],
        out_specs=pl.BlockSpec((n_b, 2 * a), lambda i, b: (0, 0)),
        scratch_shapes=[pltpu.VMEM((n_b, 2 * a), jnp.float32)],
    )
    return pl.pallas_call(
        _mlp_kern,
        grid_spec=grid_spec,
        out_shape=jax.ShapeDtypeStruct((n_b, 2 * a), jnp.float32),
        compiler_params=pltpu.CompilerParams(
            dimension_semantics=("arbitrary",)),
    )(bounds_flat, xs, invp, linear1, linear2, ml, lsl, mb3, lsb3)


def _sc_scatter_rows(data, invp_row):
    n_b, d = data.shape
    info = plsc.get_sparse_core_info()
    nw = info.num_cores * info.num_subcores
    bw = n_b // nw
    mesh = plsc.VectorSubcoreMesh(core_axis_name="c", subcore_axis_name="s")

    @functools.partial(
        pl.kernel, mesh=mesh,
        out_type=jax.ShapeDtypeStruct((n_b, d), jnp.float32),
        scratch_types=[pltpu.VMEM((bw,), jnp.int32),
                       pltpu.VMEM((bw, d), jnp.float32),
                       pltpu.SemaphoreType.DMA])
    def scatter_k(data_hbm, invp_hbm, out_hbm, idx_v, rows_v, sem):
        wid = lax.axis_index("s") * info.num_cores + lax.axis_index("c")
        base = wid * bw
        pltpu.sync_copy(invp_hbm.at[pl.ds(base, bw)], idx_v)
        pltpu.sync_copy(data_hbm.at[pl.ds(base, bw)], rows_v)
        pltpu.async_copy(rows_v, out_hbm.at[idx_v], sem).wait()

    return scatter_k(data, invp_row)


def kernel(state, option, linear1, linear2, mean_linear, log_std_linear,
           mean_bias, log_std_bias):
    n_b, _ = state.shape
    n_o = linear1.shape[0]
    a = mean_bias.shape[1]
    opt_bcast = jnp.broadcast_to(
        option.astype(jnp.int32).reshape(n_b, 1), (n_b, n_o))
    mb3 = mean_bias.reshape(n_o, 1, a)
    lsb3 = log_std_bias.reshape(n_o, 1, a)

    invp, bounds = _tc_route(opt_bcast)
    invp_row = invp.reshape(n_b)
    bounds_flat = bounds.reshape(2 * n_o)

    sorted_state = _sc_scatter_rows(state, invp_row)
    final = _tc_mlp(bounds_flat, sorted_state, invp, linear1, linear2,
                    mean_linear, log_std_linear, mb3, lsb3)
    return final[:, :a], final[:, a:2 * a]


# speedup vs baseline: 3.8326x; 1.7654x over previous
import jax
import jax.numpy as jnp
from jax.experimental import pallas as pl
from jax.experimental.pallas import tpu as pltpu


def _moe_kern(opt_ref, state_ref, l1_ref, l2_ref, ml_ref, lsl_ref, mb_ref,
              lsb_ref, mean_out, ls_out):
    o = pl.program_id(0)
    x = state_ref[...]
    h1 = jnp.maximum(
        jnp.dot(x, l1_ref[0], preferred_element_type=jnp.float32), 0.0)
    h2 = jnp.maximum(
        jnp.dot(h1, l2_ref[0], preferred_element_type=jnp.float32), 0.0)
    mean_o = jnp.dot(h2, ml_ref[0], preferred_element_type=jnp.float32)
    mean_o = mean_o + mb_ref[0]
    ls_o = jnp.dot(h2, lsl_ref[0], preferred_element_type=jnp.float32)
    ls_o = ls_o + lsb_ref[0]
    ls_o = jnp.clip(ls_o, -20.0, 2.0)
    mask = opt_ref[...] == o

    @pl.when(o == 0)
    def _():
        mean_out[...] = jnp.zeros_like(mean_out)
        ls_out[...] = jnp.zeros_like(ls_out)

    mean_out[...] = jnp.where(mask, mean_o, mean_out[...])
    ls_out[...] = jnp.where(mask, ls_o, ls_out[...])


def kernel(state, option, linear1, linear2, mean_linear, log_std_linear,
           mean_bias, log_std_bias):
    B, I = state.shape
    O, _, H = linear1.shape
    A = mean_bias.shape[1]
    Hc = linear2.shape[2]
    opt = jnp.broadcast_to(option.astype(jnp.int32).reshape(B, 1), (B, A))
    mb3 = mean_bias.reshape(O, 1, A)
    lsb3 = log_std_bias.reshape(O, 1, A)

    out_shape = (jax.ShapeDtypeStruct((B, A), jnp.float32),
                 jax.ShapeDtypeStruct((B, A), jnp.float32))
    const2 = lambda o: (0, 0)
    mean, log_std = pl.pallas_call(
        _moe_kern,
        grid=(O,),
        in_specs=[
            pl.BlockSpec((B, A), const2),
            pl.BlockSpec((B, I), const2),
            pl.BlockSpec((1, I, H), lambda o: (o, 0, 0)),
            pl.BlockSpec((1, H, Hc), lambda o: (o, 0, 0)),
            pl.BlockSpec((1, Hc, A), lambda o: (o, 0, 0)),
            pl.BlockSpec((1, Hc, A), lambda o: (o, 0, 0)),
            pl.BlockSpec((1, 1, A), lambda o: (o, 0, 0)),
            pl.BlockSpec((1, 1, A), lambda o: (o, 0, 0)),
        ],
        out_specs=(pl.BlockSpec((B, A), const2),
                   pl.BlockSpec((B, A), const2)),
        out_shape=out_shape,
        compiler_params=pltpu.CompilerParams(
            dimension_semantics=("arbitrary",)),
    )(opt, state, linear1, linear2, mean_linear, log_std_linear,
      mb3, lsb3)
    return (mean, log_std)
